# Initial kernel scaffold; baseline (speedup 1.0000x reference)
import jax
import jax.numpy as jnp
from jax.experimental import pallas as pl


def kernel(signal, edge_index, edge_weight, mask, W1, b1, gamma, beta, run_mean, run_var, W2, b2, Wl, bl, Wr, br, We, att, gat_bias, W3, b3, W4, b4):
    raise NotImplementedError("write your pallas kernel here")


# speedup vs baseline: 99.4600x; 99.4600x over previous
import jax
import jax.numpy as jnp
from jax import lax
from jax.experimental import pallas as pl
from jax.experimental.pallas import tpu as pltpu
from jax.experimental.pallas import tpu_sc as plsc

_EPS = 1e-5
_HW = 16


def _pre_body(sig_ref, nm_ref, W1_ref, b1_ref, gamma_ref, beta_ref, rm_ref,
              rv_ref, W2_ref, b2_ref, Wl0_ref, bl0_ref, Wl1_ref, bl1_ref,
              Wr0_ref, br0_ref, Wr1_ref, br1_ref, xlh_ref, xrh_ref):
    x = sig_ref[...] * nm_ref[...]
    s = gamma_ref[...] * lax.rsqrt(rv_ref[...] + _EPS)
    c = (b1_ref[...] - rm_ref[...]) * s + beta_ref[...]
    h = jnp.dot(x, W1_ref[...], preferred_element_type=jnp.float32)
    h = jnp.maximum(h * s + c, 0.0)
    h = jnp.dot(h, W2_ref[...], preferred_element_type=jnp.float32) + b2_ref[...]
    xlh_ref[0] = jnp.dot(h, Wl0_ref[...], preferred_element_type=jnp.float32) + bl0_ref[...]
    xlh_ref[1] = jnp.dot(h, Wl1_ref[...], preferred_element_type=jnp.float32) + bl1_ref[...]
    xrh_ref[0] = jnp.dot(h, Wr0_ref[...], preferred_element_type=jnp.float32) + br0_ref[...]
    xrh_ref[1] = jnp.dot(h, Wr1_ref[...], preferred_element_type=jnp.float32) + br1_ref[...]


def _run_pre(signal, nm, W1, b1, gamma, beta, rm, rv, W2, b2, wl0, bl0, wl1,
             bl1, wr0, br0, wr1, br1, tn):
    n, ch = signal.shape
    grid = n // tn

    def full(a):
        shp = a.shape
        return pl.BlockSpec(shp, lambda i, _s=len(shp): (0,) * _s)

    def row(w):
        return pl.BlockSpec((tn, w), lambda i: (i, 0))

    hrow = pl.BlockSpec((2, tn, _HW), lambda i: (0, i, 0))

    return pl.pallas_call(
        _pre_body,
        grid=(grid,),
        in_specs=[row(ch), row(1), full(W1), full(b1), full(gamma), full(beta),
                  full(rm), full(rv), full(W2), full(b2), full(wl0), full(bl0),
                  full(wl1), full(bl1), full(wr0), full(br0), full(wr1),
                  full(br1)],
        out_specs=[hrow, hrow],
        out_shape=[jax.ShapeDtypeStruct((2, n, _HW), jnp.float32)] * 2,
    )(signal, nm, W1, b1, gamma, beta, rm, rv, W2, b2, wl0, bl0, wl1, bl1,
      wr0, br0, wr1, br1)


def _make_sc_edge(n, e, hid):
    nc, ns = 2, 16
    tpe = e // ns
    sub = 80
    nsub = 2
    cb = sub * nsub
    nch = tpe // cb
    stripe = (-(-n // ns) + 7) // 8 * 8
    last = n - stripe * (ns - 1)
    zr = 136
    assert n % 8 == 0 and last > 0 and last % 8 == 0
    assert tpe % cb == 0 and nch >= 4

    mesh = plsc.VectorSubcoreMesh(core_axis_name="c", subcore_axis_name="s")

    def body(xl0_hbm, xl1_hbm, xr0_hbm, xr1_hbm, src_hbm, dst_hbm, ew_hbm,
             wes0_hbm, wes1_hbm, atts0_hbm, atts1_hbm,
             out_hbm, acc_sh, wes_v, atts_v,
             srcA0, srcB0, dstA0, dstB0, ewv0, xlr0, xrr0, valb0, dlA0, dlB0,
             srcA1, srcB1, dstA1, dstB1, ewv1, xlr1, xrr1, valb1, dlA1, dlB1,
             zrow, esem0, gsem0, esem1, gsem1):
        cid = lax.axis_index("c")
        sid = lax.axis_index("s")
        ebase = sid * tpe

        @pl.when(cid == 0)
        def _():
            pltpu.sync_copy(wes0_hbm, wes_v)
            pltpu.sync_copy(atts0_hbm, atts_v)

        @pl.when(cid == 1)
        def _():
            pltpu.sync_copy(wes1_hbm, wes_v)
            pltpu.sync_copy(atts1_hbm, atts_v)

        z16 = jnp.zeros((16,), jnp.float32)

        def zfill(i, _):
            zrow[i, pl.ds(0, 16)] = z16
            return 0

        lax.fori_loop(0, zr, zfill, 0)

        def vfill(i, _):
            valb0[i, pl.ds(0, 16)] = z16
            valb1[i, pl.ds(0, 16)] = z16
            return 0

        lax.fori_loop(0, cb, vfill, 0)

        start = sid * stripe

        def zero_rows(count):
            nfull, rem = divmod(count, zr)

            def zcp(j, _):
                pltpu.sync_copy(zrow, acc_sh.at[pl.ds(start + j * zr, zr)])
                return 0

            lax.fori_loop(0, nfull, zcp, 0)
            if rem:
                pltpu.sync_copy(
                    zrow.at[pl.ds(0, rem)],
                    acc_sh.at[pl.ds(start + nfull * zr, rem)])

        @pl.when(sid < ns - 1)
        def _():
            zero_rows(stripe)

        @pl.when(sid == ns - 1)
        def _():
            zero_rows(last)

        plsc.subcore_barrier()

        bufs = ((srcA0, srcB0, dstA0, dstB0, ewv0, xlr0, xrr0, valb0, dlA0,
                 dlB0, esem0, gsem0),
                (srcA1, srcB1, dstA1, dstB1, ewv1, xlr1, xrr1, valb1, dlA1,
                 dlB1, esem1, gsem1))

        def issue_edge(ci, B):
            eb = ebase + ci * cb
            pltpu.async_copy(src_hbm.at[pl.ds(eb, sub)], B[0], B[10])
            pltpu.async_copy(src_hbm.at[pl.ds(eb + sub, sub)], B[1], B[10])
            pltpu.async_copy(dst_hbm.at[pl.ds(eb, sub)], B[2], B[10])
            pltpu.async_copy(dst_hbm.at[pl.ds(eb + sub, sub)], B[3], B[10])
            pltpu.async_copy(ew_hbm.at[pl.ds(eb, cb)], B[4], B[10])

        def wait_edge(B):
            for r in (B[0], B[1], B[2], B[3]):
                pltpu.make_async_copy(
                    src_hbm.at[pl.ds(0, sub)], r, B[10]).wait()
            pltpu.make_async_copy(ew_hbm.at[pl.ds(0, cb)], B[4], B[10]).wait()

        def issue_gather(B):
            @pl.when(cid == 0)
            def _():
                pltpu.async_copy(xl0_hbm.at[B[0]], B[5].at[pl.ds(0, sub)], B[11])
                pltpu.async_copy(xl0_hbm.at[B[1]], B[5].at[pl.ds(sub, sub)], B[11])
                pltpu.async_copy(xr0_hbm.at[B[2]], B[6].at[pl.ds(0, sub)], B[11])
                pltpu.async_copy(xr0_hbm.at[B[3]], B[6].at[pl.ds(sub, sub)], B[11])

            @pl.when(cid == 1)
            def _():
                pltpu.async_copy(xl1_hbm.at[B[0]], B[5].at[pl.ds(0, sub)], B[11])
                pltpu.async_copy(xl1_hbm.at[B[1]], B[5].at[pl.ds(sub, sub)], B[11])
                pltpu.async_copy(xr1_hbm.at[B[2]], B[6].at[pl.ds(0, sub)], B[11])
                pltpu.async_copy(xr1_hbm.at[B[3]], B[6].at[pl.ds(sub, sub)], B[11])

        def wait_gather(B):
            pltpu.make_async_copy(
                xl0_hbm.at[B[0]], B[5].at[pl.ds(0, sub)], B[11]).wait()
            pltpu.make_async_copy(
                xl0_hbm.at[B[1]], B[5].at[pl.ds(sub, sub)], B[11]).wait()
            pltpu.make_async_copy(
                xr0_hbm.at[B[2]], B[6].at[pl.ds(0, sub)], B[11]).wait()
            pltpu.make_async_copy(
                xr0_hbm.at[B[3]], B[6].at[pl.ds(sub, sub)], B[11]).wait()

        def compute_scatter(B):
            ewv, xlr, xrr, valb = B[4], B[5], B[6], B[7]
            for k in range(nsub):
                dstv = B[2 + k]
                for gg in range(sub // 16):
                    r0 = k * sub + gg * 16
                    rows = lax.iota(jnp.int32, 16) + r0
                    dstg = dstv[pl.ds(gg * 16, 16)]
                    ewg = ewv[pl.ds(r0, 16)]
                    a0 = jnp.zeros((16,), jnp.float32)
                    xls = []
                    for comp in range(hid):
                        cix = jnp.full((16,), comp, jnp.int32)
                        xlc = plsc.load_gather(xlr, [rows, cix])
                        xrc = plsc.load_gather(xrr, [rows, cix])
                        xls.append(xlc)
                        t = xlc + xrc + ewg * wes_v[comp]
                        t = jnp.maximum(t, 0.2 * t)
                        a0 = a0 + t * atts_v[comp]
                    ex = jnp.exp(jnp.clip(a0, -60.0, 60.0))
                    for comp in range(hid):
                        cix = jnp.full((16,), comp, jnp.int32)
                        plsc.store_scatter(valb, [rows, cix], xls[comp] * ex)
                    plsc.store_scatter(
                        valb, [rows, jnp.full((16,), hid, jnp.int32)], ex)
                    B[8 + k][pl.ds(gg * 16, 16)] = dstg
            pltpu.sync_copy(valb.at[pl.ds(0, sub)], acc_sh.at[B[8]], add=True)
            pltpu.sync_copy(valb.at[pl.ds(sub, sub)], acc_sh.at[B[9]], add=True)

        issue_edge(0, bufs[0])
        wait_edge(bufs[0])
        issue_gather(bufs[0])
        issue_edge(1, bufs[1])

        def step(ci, B, Bo):
            @pl.when(ci + 1 < nch)
            def _():
                wait_edge(Bo)
                issue_gather(Bo)

            wait_gather(B)
            compute_scatter(B)

            @pl.when(ci + 2 < nch)
            def _():
                issue_edge(ci + 2, B)

        def chunk_body(ci, _):
            @pl.when(ci % 2 == 0)
            def _():
                step(ci, bufs[0], bufs[1])

            @pl.when(ci % 2 == 1)
            def _():
                step(ci, bufs[1], bufs[0])

            return 0

        lax.fori_loop(0, nch, chunk_body, 0)

        plsc.subcore_barrier()
        ostart = cid * n + start

        @pl.when(sid < ns - 1)
        def _():
            pltpu.sync_copy(acc_sh.at[pl.ds(start, stripe)],
                            out_hbm.at[pl.ds(ostart, stripe)])

        @pl.when(sid == ns - 1)
        def _():
            pltpu.sync_copy(acc_sh.at[pl.ds(start, last)],
                            out_hbm.at[pl.ds(ostart, last)])

    def bufset():
        return [
            pltpu.VMEM((sub,), jnp.int32),
            pltpu.VMEM((sub,), jnp.int32),
            pltpu.VMEM((sub,), jnp.int32),
            pltpu.VMEM((sub,), jnp.int32),
            pltpu.VMEM((cb,), jnp.float32),
            pltpu.VMEM((cb, _HW), jnp.float32),
            pltpu.VMEM((cb, _HW), jnp.float32),
            pltpu.VMEM((cb, _HW), jnp.float32),
            pltpu.VMEM((sub,), jnp.int32),
            pltpu.VMEM((sub,), jnp.int32),
        ]

    return pl.kernel(
        body,
        out_type=jax.ShapeDtypeStruct((2 * n, _HW), jnp.float32),
        mesh=mesh,
        compiler_params=pltpu.CompilerParams(
            needs_layout_passes=False, use_tc_tiling_on_sc=False),
        scratch_types=[
            pltpu.VMEM_SHARED((n, _HW), jnp.float32),
            pltpu.VMEM((_HW, 16), jnp.float32),
            pltpu.VMEM((_HW, 16), jnp.float32),
            *bufset(),
            *bufset(),
            pltpu.VMEM((zr, _HW), jnp.float32),
            pltpu.SemaphoreType.DMA,
            pltpu.SemaphoreType.DMA,
            pltpu.SemaphoreType.DMA,
            pltpu.SemaphoreType.DMA,
        ],
    )


def _post_body(acc_ref, gb0_ref, gb1_ref, W3a_ref, W3b_ref, b3_ref, W4_ref,
               b4_ref, out_ref):
    o0 = acc_ref[0]
    o1 = acc_ref[1]
    tn = o0.shape[0]
    d0 = jnp.broadcast_to(o0[:, 10:11], (tn, _HW)) + 1e-16
    d1 = jnp.broadcast_to(o1[:, 10:11], (tn, _HW)) + 1e-16
    g0 = o0 / d0 + gb0_ref[...]
    g1 = o1 / d1 + gb1_ref[...]
    h = (jnp.dot(g0, W3a_ref[...], preferred_element_type=jnp.float32)
         + jnp.dot(g1, W3b_ref[...], preferred_element_type=jnp.float32)
         + b3_ref[...])
    out_ref[...] = jnp.dot(h, W4_ref[...], preferred_element_type=jnp.float32) + b4_ref[...]


def _run_post(acc, gb0, gb1, W3a, W3b, b3, W4, b4, tn):
    n = acc.shape[1]
    ch = W4.shape[1]
    grid = n // tn

    def full(a):
        shp = a.shape
        return pl.BlockSpec(shp, lambda i, _s=len(shp): (0,) * _s)

    return pl.pallas_call(
        _post_body,
        grid=(grid,),
        in_specs=[pl.BlockSpec((2, tn, _HW), lambda i: (0, i, 0)), full(gb0),
                  full(gb1), full(W3a), full(W3b), full(b3), full(W4),
                  full(b4)],
        out_specs=pl.BlockSpec((tn, ch), lambda i: (i, 0)),
        out_shape=jax.ShapeDtypeStruct((n, ch), jnp.float32),
    )(acc, gb0, gb1, W3a, W3b, b3, W4, b4)


def kernel(signal, edge_index, edge_weight, mask, W1, b1, gamma, beta,
           run_mean, run_var, W2, b2, Wl, bl, Wr, br, We, att, gat_bias,
           W3, b3, W4, b4):
    f32 = jnp.float32
    n, ch = signal.shape
    e = edge_index.shape[1]
    hid = W2.shape[1]
    hc = Wl.shape[1]
    nheads = att.shape[0]
    hd = hc // nheads

    nm = jnp.logical_not(mask).astype(f32)[:, None]
    src = edge_index[0]
    dst = edge_index[1]
    ew = edge_weight[:, 0]

    def headw(W, h):
        return jnp.zeros((hid, _HW), f32).at[:, :hd].set(W[:, h * hd:(h + 1) * hd])

    def headb(v, h):
        return jnp.zeros((1, _HW), f32).at[0, :hd].set(v[h * hd:(h + 1) * hd])

    wl0, wl1 = headw(Wl, 0), headw(Wl, 1)
    wr0, wr1 = headw(Wr, 0), headw(Wr, 1)
    bl0, bl1 = headb(bl, 0), headb(bl, 1)
    br0, br1 = headb(br, 0), headb(br, 1)

    wesh = jnp.zeros((nheads, _HW, 16), f32)
    attsh = jnp.zeros((nheads, _HW, 16), f32)
    for h in range(nheads):
        wesh = wesh.at[h, :hd, :].set(
            jnp.broadcast_to(We[0, h * hd:(h + 1) * hd][:, None], (hd, 16)))
        attsh = attsh.at[h, :hd, :].set(
            jnp.broadcast_to(att[h][:, None], (hd, 16)))

    gb0, gb1 = headb(gat_bias, 0), headb(gat_bias, 1)
    W3a = jnp.zeros((_HW, W3.shape[1]), f32).at[:hd, :].set(W3[:hd, :])
    W3b = jnp.zeros((_HW, W3.shape[1]), f32).at[:hd, :].set(W3[hd:, :])

    tn = 2000
    xlh, xrh = _run_pre(
        signal, nm, W1, b1.reshape(1, -1), gamma.reshape(1, -1),
        beta.reshape(1, -1), run_mean.reshape(1, -1), run_var.reshape(1, -1),
        W2, b2.reshape(1, -1), wl0, bl0, wl1, bl1, wr0, br0, wr1, br1, tn)

    acc = _make_sc_edge(n, e, hd)(
        xlh[0], xlh[1], xrh[0], xrh[1], src, dst, ew,
        wesh[0], wesh[1], attsh[0], attsh[1])

    return _run_post(acc.reshape(2, n, _HW), gb0, gb1, W3a, W3b,
                     b3.reshape(1, -1), W4, b4.reshape(1, -1), tn)


# speedup vs baseline: 109.1408x; 1.0973x over previous
import jax
import jax.numpy as jnp
from jax import lax
from jax.experimental import pallas as pl
from jax.experimental.pallas import tpu as pltpu
from jax.experimental.pallas import tpu_sc as plsc

_EPS = 1e-5
_HW = 16


def _pre_body(sig_ref, nm_ref, W1_ref, b1_ref, gamma_ref, beta_ref, rm_ref,
              rv_ref, W2_ref, b2_ref, Wl0_ref, bl0_ref, Wl1_ref, bl1_ref,
              Wr0_ref, br0_ref, Wr1_ref, br1_ref, xl0_ref, xl1_ref, xr0_ref,
              xr1_ref):
    x = sig_ref[...] * nm_ref[...]
    s = gamma_ref[...] * lax.rsqrt(rv_ref[...] + _EPS)
    c = (b1_ref[...] - rm_ref[...]) * s + beta_ref[...]
    h = jnp.dot(x, W1_ref[...], preferred_element_type=jnp.float32)
    h = jnp.maximum(h * s + c, 0.0)
    h = jnp.dot(h, W2_ref[...], preferred_element_type=jnp.float32) + b2_ref[...]
    xl0_ref[...] = jnp.dot(h, Wl0_ref[...], preferred_element_type=jnp.float32) + bl0_ref[...]
    xl1_ref[...] = jnp.dot(h, Wl1_ref[...], preferred_element_type=jnp.float32) + bl1_ref[...]
    xr0_ref[...] = jnp.dot(h, Wr0_ref[...], preferred_element_type=jnp.float32) + br0_ref[...]
    xr1_ref[...] = jnp.dot(h, Wr1_ref[...], preferred_element_type=jnp.float32) + br1_ref[...]


def _run_pre(signal, nm, W1, b1, gamma, beta, rm, rv, W2, b2, wl0, bl0, wl1,
             bl1, wr0, br0, wr1, br1, tn):
    n, ch = signal.shape
    grid = n // tn

    def full(a):
        shp = a.shape
        return pl.BlockSpec(shp, lambda i, _s=len(shp): (0,) * _s)

    def row(w):
        return pl.BlockSpec((tn, w), lambda i: (i, 0))

    return pl.pallas_call(
        _pre_body,
        grid=(grid,),
        in_specs=[row(ch), row(1), full(W1), full(b1), full(gamma), full(beta),
                  full(rm), full(rv), full(W2), full(b2), full(wl0), full(bl0),
                  full(wl1), full(bl1), full(wr0), full(br0), full(wr1),
                  full(br1)],
        out_specs=[row(_HW)] * 4,
        out_shape=[jax.ShapeDtypeStruct((n, _HW), jnp.float32)] * 4,
    )(signal, nm, W1, b1, gamma, beta, rm, rv, W2, b2, wl0, bl0, wl1, bl1,
      wr0, br0, wr1, br1)


def _make_sc_edge(n, e, hid):
    nc, ns = 2, 16
    tpe = e // ns
    sub = 80
    nsub = 2
    cb = sub * nsub
    nch = tpe // cb
    stripe = (-(-n // ns) + 7) // 8 * 8
    last = n - stripe * (ns - 1)
    zr = 624
    assert n % 8 == 0 and last > 0 and last % 8 == 0
    assert tpe % cb == 0 and nch >= 4

    mesh = plsc.VectorSubcoreMesh(core_axis_name="c", subcore_axis_name="s")

    def body(xl0_hbm, xl1_hbm, xr0_hbm, xr1_hbm, src_hbm, dst_hbm, ew_hbm,
             wes0_hbm, wes1_hbm, atts0_hbm, atts1_hbm,
             out_hbm, acc_sh, wes_v, atts_v,
             srcA0, srcB0, dstA0, dstB0, ewv0, xlr0, xrr0, valb0, dlA0, dlB0,
             srcA1, srcB1, dstA1, dstB1, ewv1, xlr1, xrr1, valb1, dlA1, dlB1,
             zrow, esem0, gsem0, esem1, gsem1):
        cid = lax.axis_index("c")
        sid = lax.axis_index("s")
        ebase = sid * tpe

        @pl.when(cid == 0)
        def _():
            pltpu.sync_copy(wes0_hbm, wes_v)
            pltpu.sync_copy(atts0_hbm, atts_v)

        @pl.when(cid == 1)
        def _():
            pltpu.sync_copy(wes1_hbm, wes_v)
            pltpu.sync_copy(atts1_hbm, atts_v)

        z16 = jnp.zeros((16,), jnp.float32)

        def zfill(i, _):
            zrow[i, pl.ds(0, 16)] = z16
            return 0

        lax.fori_loop(0, zr, zfill, 0)

        def vfill(i, _):
            valb0[i, pl.ds(0, 16)] = z16
            valb1[i, pl.ds(0, 16)] = z16
            return 0

        lax.fori_loop(0, cb, vfill, 0)

        start = sid * stripe

        def zero_rows(count):
            nfull, rem = divmod(count, zr)

            def zcp(j, _):
                pltpu.sync_copy(zrow, acc_sh.at[pl.ds(start + j * zr, zr)])
                return 0

            lax.fori_loop(0, nfull, zcp, 0)
            if rem:
                pltpu.sync_copy(
                    zrow.at[pl.ds(0, rem)],
                    acc_sh.at[pl.ds(start + nfull * zr, rem)])

        @pl.when(sid < ns - 1)
        def _():
            zero_rows(stripe)

        @pl.when(sid == ns - 1)
        def _():
            zero_rows(last)

        plsc.subcore_barrier()

        bufs = ((srcA0, srcB0, dstA0, dstB0, ewv0, xlr0, xrr0, valb0, dlA0,
                 dlB0, esem0, gsem0),
                (srcA1, srcB1, dstA1, dstB1, ewv1, xlr1, xrr1, valb1, dlA1,
                 dlB1, esem1, gsem1))

        def issue_edge(ci, B):
            eb = ebase + ci * cb
            pltpu.async_copy(src_hbm.at[pl.ds(eb, sub)], B[0], B[10])
            pltpu.async_copy(src_hbm.at[pl.ds(eb + sub, sub)], B[1], B[10])
            pltpu.async_copy(dst_hbm.at[pl.ds(eb, sub)], B[2], B[10])
            pltpu.async_copy(dst_hbm.at[pl.ds(eb + sub, sub)], B[3], B[10])
            pltpu.async_copy(ew_hbm.at[pl.ds(eb, cb)], B[4], B[10])

        def wait_edge(B):
            for r in (B[0], B[1], B[2], B[3]):
                pltpu.make_async_copy(
                    src_hbm.at[pl.ds(0, sub)], r, B[10]).wait()
            pltpu.make_async_copy(ew_hbm.at[pl.ds(0, cb)], B[4], B[10]).wait()

        def issue_gather(B):
            @pl.when(cid == 0)
            def _():
                pltpu.async_copy(xl0_hbm.at[B[0]], B[5].at[pl.ds(0, sub)], B[11])
                pltpu.async_copy(xl0_hbm.at[B[1]], B[5].at[pl.ds(sub, sub)], B[11])
                pltpu.async_copy(xr0_hbm.at[B[2]], B[6].at[pl.ds(0, sub)], B[11])
                pltpu.async_copy(xr0_hbm.at[B[3]], B[6].at[pl.ds(sub, sub)], B[11])

            @pl.when(cid == 1)
            def _():
                pltpu.async_copy(xl1_hbm.at[B[0]], B[5].at[pl.ds(0, sub)], B[11])
                pltpu.async_copy(xl1_hbm.at[B[1]], B[5].at[pl.ds(sub, sub)], B[11])
                pltpu.async_copy(xr1_hbm.at[B[2]], B[6].at[pl.ds(0, sub)], B[11])
                pltpu.async_copy(xr1_hbm.at[B[3]], B[6].at[pl.ds(sub, sub)], B[11])

        def wait_gather(B):
            pltpu.make_async_copy(
                xl0_hbm.at[B[0]], B[5].at[pl.ds(0, sub)], B[11]).wait()
            pltpu.make_async_copy(
                xl0_hbm.at[B[1]], B[5].at[pl.ds(sub, sub)], B[11]).wait()
            pltpu.make_async_copy(
                xr0_hbm.at[B[2]], B[6].at[pl.ds(0, sub)], B[11]).wait()
            pltpu.make_async_copy(
                xr0_hbm.at[B[3]], B[6].at[pl.ds(sub, sub)], B[11]).wait()

        def compute_scatter(B):
            ewv, xlr, xrr, valb = B[4], B[5], B[6], B[7]
            for k in range(nsub):
                dstv = B[2 + k]
                for gg in range(sub // 16):
                    r0 = k * sub + gg * 16
                    rows = lax.iota(jnp.int32, 16) + r0
                    dstg = dstv[pl.ds(gg * 16, 16)]
                    ewg = ewv[pl.ds(r0, 16)]
                    a0 = jnp.zeros((16,), jnp.float32)
                    xls = []
                    for comp in range(hid):
                        cix = jnp.full((16,), comp, jnp.int32)
                        xlc = plsc.load_gather(xlr, [rows, cix])
                        xrc = plsc.load_gather(xrr, [rows, cix])
                        xls.append(xlc)
                        t = xlc + xrc + ewg * wes_v[comp]
                        t = jnp.maximum(t, 0.2 * t)
                        a0 = a0 + t * atts_v[comp]
                    ex = jnp.exp(jnp.clip(a0, -60.0, 60.0))
                    for comp in range(hid):
                        cix = jnp.full((16,), comp, jnp.int32)
                        plsc.store_scatter(valb, [rows, cix], xls[comp] * ex)
                    plsc.store_scatter(
                        valb, [rows, jnp.full((16,), hid, jnp.int32)], ex)
                    B[8 + k][pl.ds(gg * 16, 16)] = dstg
            cp1 = pltpu.async_copy(
                valb.at[pl.ds(0, sub)], acc_sh.at[B[8]], B[10], add=True)
            cp2 = pltpu.async_copy(
                valb.at[pl.ds(sub, sub)], acc_sh.at[B[9]], B[11], add=True)
            cp1.wait()
            cp2.wait()

        issue_edge(0, bufs[0])
        wait_edge(bufs[0])
        issue_gather(bufs[0])
        issue_edge(1, bufs[1])

        def step(ci, B, Bo):
            @pl.when(ci + 1 < nch)
            def _():
                wait_edge(Bo)
                issue_gather(Bo)

            wait_gather(B)
            compute_scatter(B)

            @pl.when(ci + 2 < nch)
            def _():
                issue_edge(ci + 2, B)

        def chunk_body(ci, _):
            @pl.when(ci % 2 == 0)
            def _():
                step(ci, bufs[0], bufs[1])

            @pl.when(ci % 2 == 1)
            def _():
                step(ci, bufs[1], bufs[0])

            return 0

        lax.fori_loop(0, nch, chunk_body, 0)

        plsc.subcore_barrier()
        ostart = cid * n + start

        @pl.when(sid < ns - 1)
        def _():
            pltpu.sync_copy(acc_sh.at[pl.ds(start, stripe)],
                            out_hbm.at[pl.ds(ostart, stripe)])

        @pl.when(sid == ns - 1)
        def _():
            pltpu.sync_copy(acc_sh.at[pl.ds(start, last)],
                            out_hbm.at[pl.ds(ostart, last)])

    def bufset():
        return [
            pltpu.VMEM((sub,), jnp.int32),
            pltpu.VMEM((sub,), jnp.int32),
            pltpu.VMEM((sub,), jnp.int32),
            pltpu.VMEM((sub,), jnp.int32),
            pltpu.VMEM((cb,), jnp.float32),
            pltpu.VMEM((cb, _HW), jnp.float32),
            pltpu.VMEM((cb, _HW), jnp.float32),
            pltpu.VMEM((cb, _HW), jnp.float32),
            pltpu.VMEM((sub,), jnp.int32),
            pltpu.VMEM((sub,), jnp.int32),
        ]

    return pl.kernel(
        body,
        out_type=jax.ShapeDtypeStruct((2 * n, _HW), jnp.float32),
        mesh=mesh,
        compiler_params=pltpu.CompilerParams(
            needs_layout_passes=False, use_tc_tiling_on_sc=False),
        scratch_types=[
            pltpu.VMEM_SHARED((n, _HW), jnp.float32),
            pltpu.VMEM((_HW, 16), jnp.float32),
            pltpu.VMEM((_HW, 16), jnp.float32),
            *bufset(),
            *bufset(),
            pltpu.VMEM((zr, _HW), jnp.float32),
            pltpu.SemaphoreType.DMA,
            pltpu.SemaphoreType.DMA,
            pltpu.SemaphoreType.DMA,
            pltpu.SemaphoreType.DMA,
        ],
    )


def _post_body(acc_ref, gb0_ref, gb1_ref, W3a_ref, W3b_ref, b3_ref, W4_ref,
               b4_ref, out_ref):
    o0 = acc_ref[0]
    o1 = acc_ref[1]
    tn = o0.shape[0]
    d0 = jnp.broadcast_to(o0[:, 10:11], (tn, _HW)) + 1e-16
    d1 = jnp.broadcast_to(o1[:, 10:11], (tn, _HW)) + 1e-16
    g0 = o0 / d0 + gb0_ref[...]
    g1 = o1 / d1 + gb1_ref[...]
    h = (jnp.dot(g0, W3a_ref[...], preferred_element_type=jnp.float32)
         + jnp.dot(g1, W3b_ref[...], preferred_element_type=jnp.float32)
         + b3_ref[...])
    out_ref[...] = jnp.dot(h, W4_ref[...], preferred_element_type=jnp.float32) + b4_ref[...]


def _run_post(acc, gb0, gb1, W3a, W3b, b3, W4, b4, tn):
    n = acc.shape[1]
    ch = W4.shape[1]
    grid = n // tn

    def full(a):
        shp = a.shape
        return pl.BlockSpec(shp, lambda i, _s=len(shp): (0,) * _s)

    return pl.pallas_call(
        _post_body,
        grid=(grid,),
        in_specs=[pl.BlockSpec((2, tn, _HW), lambda i: (0, i, 0)), full(gb0),
                  full(gb1), full(W3a), full(W3b), full(b3), full(W4),
                  full(b4)],
        out_specs=pl.BlockSpec((tn, ch), lambda i: (i, 0)),
        out_shape=jax.ShapeDtypeStruct((n, ch), jnp.float32),
    )(acc, gb0, gb1, W3a, W3b, b3, W4, b4)


def kernel(signal, edge_index, edge_weight, mask, W1, b1, gamma, beta,
           run_mean, run_var, W2, b2, Wl, bl, Wr, br, We, att, gat_bias,
           W3, b3, W4, b4):
    f32 = jnp.float32
    n, ch = signal.shape
    e = edge_index.shape[1]
    hid = W2.shape[1]
    hc = Wl.shape[1]
    nheads = att.shape[0]
    hd = hc // nheads

    nm = jnp.logical_not(mask).astype(f32)[:, None]
    src = edge_index[0]
    dst = edge_index[1]
    ew = edge_weight[:, 0]

    def headw(W, h):
        return jnp.zeros((hid, _HW), f32).at[:, :hd].set(W[:, h * hd:(h + 1) * hd])

    def headb(v, h):
        return jnp.zeros((1, _HW), f32).at[0, :hd].set(v[h * hd:(h + 1) * hd])

    wl0, wl1 = headw(Wl, 0), headw(Wl, 1)
    wr0, wr1 = headw(Wr, 0), headw(Wr, 1)
    bl0, bl1 = headb(bl, 0), headb(bl, 1)
    br0, br1 = headb(br, 0), headb(br, 1)

    wesh = jnp.zeros((nheads, _HW, 16), f32)
    attsh = jnp.zeros((nheads, _HW, 16), f32)
    for h in range(nheads):
        wesh = wesh.at[h, :hd, :].set(
            jnp.broadcast_to(We[0, h * hd:(h + 1) * hd][:, None], (hd, 16)))
        attsh = attsh.at[h, :hd, :].set(
            jnp.broadcast_to(att[h][:, None], (hd, 16)))

    gb0, gb1 = headb(gat_bias, 0), headb(gat_bias, 1)
    W3a = jnp.zeros((_HW, W3.shape[1]), f32).at[:hd, :].set(W3[:hd, :])
    W3b = jnp.zeros((_HW, W3.shape[1]), f32).at[:hd, :].set(W3[hd:, :])

    tn = 2000
    xl0, xl1, xr0, xr1 = _run_pre(
        signal, nm, W1, b1.reshape(1, -1), gamma.reshape(1, -1),
        beta.reshape(1, -1), run_mean.reshape(1, -1), run_var.reshape(1, -1),
        W2, b2.reshape(1, -1), wl0, bl0, wl1, bl1, wr0, br0, wr1, br1, tn)

    acc = _make_sc_edge(n, e, hd)(
        xl0, xl1, xr0, xr1, src, dst, ew,
        wesh[0], wesh[1], attsh[0], attsh[1])

    return _run_post(acc.reshape(2, n, _HW), gb0, gb1, W3a, W3b,
                     b3.reshape(1, -1), W4, b4.reshape(1, -1), tn)


# speedup vs baseline: 109.1447x; 1.0000x over previous
import jax
import jax.numpy as jnp
from jax import lax
from jax.experimental import pallas as pl
from jax.experimental.pallas import tpu as pltpu
from jax.experimental.pallas import tpu_sc as plsc

_EPS = 1e-5
_HW = 16


def _pre_body(sig_ref, nm_ref, W1_ref, b1_ref, gamma_ref, beta_ref, rm_ref,
              rv_ref, W2_ref, b2_ref, Wl0_ref, bl0_ref, Wl1_ref, bl1_ref,
              Wr0_ref, br0_ref, Wr1_ref, br1_ref, xl0_ref, xl1_ref, xr0_ref,
              xr1_ref):
    x = sig_ref[...] * nm_ref[...]
    s = gamma_ref[...] * lax.rsqrt(rv_ref[...] + _EPS)
    c = (b1_ref[...] - rm_ref[...]) * s + beta_ref[...]
    h = jnp.dot(x, W1_ref[...], preferred_element_type=jnp.float32)
    h = jnp.maximum(h * s + c, 0.0)
    h = jnp.dot(h, W2_ref[...], preferred_element_type=jnp.float32) + b2_ref[...]
    xl0_ref[...] = jnp.dot(h, Wl0_ref[...], preferred_element_type=jnp.float32) + bl0_ref[...]
    xl1_ref[...] = jnp.dot(h, Wl1_ref[...], preferred_element_type=jnp.float32) + bl1_ref[...]
    xr0_ref[...] = jnp.dot(h, Wr0_ref[...], preferred_element_type=jnp.float32) + br0_ref[...]
    xr1_ref[...] = jnp.dot(h, Wr1_ref[...], preferred_element_type=jnp.float32) + br1_ref[...]


def _run_pre(signal, nm, W1, b1, gamma, beta, rm, rv, W2, b2, wl0, bl0, wl1,
             bl1, wr0, br0, wr1, br1, tn):
    n, ch = signal.shape
    grid = n // tn

    def full(a):
        shp = a.shape
        return pl.BlockSpec(shp, lambda i, _s=len(shp): (0,) * _s)

    def row(w):
        return pl.BlockSpec((tn, w), lambda i: (i, 0))

    return pl.pallas_call(
        _pre_body,
        grid=(grid,),
        in_specs=[row(ch), row(1), full(W1), full(b1), full(gamma), full(beta),
                  full(rm), full(rv), full(W2), full(b2), full(wl0), full(bl0),
                  full(wl1), full(bl1), full(wr0), full(br0), full(wr1),
                  full(br1)],
        out_specs=[row(_HW)] * 4,
        out_shape=[jax.ShapeDtypeStruct((n, _HW), jnp.float32)] * 4,
    )(signal, nm, W1, b1, gamma, beta, rm, rv, W2, b2, wl0, bl0, wl1, bl1,
      wr0, br0, wr1, br1)


def _make_sc_edge(n, e, hid):
    nc, ns = 2, 16
    tpe = e // ns
    sub = 80
    nsub = 2
    cb = sub * nsub
    nch = tpe // cb
    stripe = (-(-n // ns) + 7) // 8 * 8
    last = n - stripe * (ns - 1)
    zr = 624
    assert n % 8 == 0 and last > 0 and last % 8 == 0
    assert tpe % cb == 0 and nch >= 4

    mesh = plsc.VectorSubcoreMesh(core_axis_name="c", subcore_axis_name="s")

    def body(xl0_hbm, xl1_hbm, xr0_hbm, xr1_hbm, src_hbm, dst_hbm, ew_hbm,
             wes0_hbm, wes1_hbm, atts0_hbm, atts1_hbm,
             out_hbm, acc_sh, wes_v, atts_v,
             srcA0, srcB0, dstA0, dstB0, ewv0, xlr0, xrr0, valb0,
             srcA1, srcB1, dstA1, dstB1, ewv1, xlr1, xrr1, valb1,
             zrow, esem0, gsem0, esem1, gsem1):
        cid = lax.axis_index("c")
        sid = lax.axis_index("s")
        ebase = sid * tpe

        @pl.when(cid == 0)
        def _():
            pltpu.sync_copy(wes0_hbm, wes_v)
            pltpu.sync_copy(atts0_hbm, atts_v)

        @pl.when(cid == 1)
        def _():
            pltpu.sync_copy(wes1_hbm, wes_v)
            pltpu.sync_copy(atts1_hbm, atts_v)

        z16 = jnp.zeros((16,), jnp.float32)

        def zfill(i, _):
            zrow[i, pl.ds(0, 16)] = z16
            return 0

        lax.fori_loop(0, zr, zfill, 0)

        def vfill(i, _):
            valb0[i, pl.ds(0, 16)] = z16
            valb1[i, pl.ds(0, 16)] = z16
            return 0

        lax.fori_loop(0, cb, vfill, 0)

        start = sid * stripe

        def zero_rows(count):
            nfull, rem = divmod(count, zr)

            def zcp(j, _):
                pltpu.sync_copy(zrow, acc_sh.at[pl.ds(start + j * zr, zr)])
                return 0

            lax.fori_loop(0, nfull, zcp, 0)
            if rem:
                pltpu.sync_copy(
                    zrow.at[pl.ds(0, rem)],
                    acc_sh.at[pl.ds(start + nfull * zr, rem)])

        @pl.when(sid < ns - 1)
        def _():
            zero_rows(stripe)

        @pl.when(sid == ns - 1)
        def _():
            zero_rows(last)

        plsc.subcore_barrier()

        bufs = ((srcA0, srcB0, dstA0, dstB0, ewv0, xlr0, xrr0, valb0,
                 esem0, gsem0),
                (srcA1, srcB1, dstA1, dstB1, ewv1, xlr1, xrr1, valb1,
                 esem1, gsem1))

        def issue_edge(ci, B):
            eb = ebase + ci * cb
            pltpu.async_copy(src_hbm.at[pl.ds(eb, sub)], B[0], B[8])
            pltpu.async_copy(src_hbm.at[pl.ds(eb + sub, sub)], B[1], B[8])
            pltpu.async_copy(dst_hbm.at[pl.ds(eb, sub)], B[2], B[8])
            pltpu.async_copy(dst_hbm.at[pl.ds(eb + sub, sub)], B[3], B[8])
            pltpu.async_copy(ew_hbm.at[pl.ds(eb, cb)], B[4], B[8])

        def wait_edge(B):
            for r in (B[0], B[1], B[2], B[3]):
                pltpu.make_async_copy(
                    src_hbm.at[pl.ds(0, sub)], r, B[8]).wait()
            pltpu.make_async_copy(ew_hbm.at[pl.ds(0, cb)], B[4], B[8]).wait()

        def issue_gather(B):
            @pl.when(cid == 0)
            def _():
                pltpu.async_copy(xl0_hbm.at[B[0]], B[5].at[pl.ds(0, sub)], B[9])
                pltpu.async_copy(xl0_hbm.at[B[1]], B[5].at[pl.ds(sub, sub)], B[9])
                pltpu.async_copy(xr0_hbm.at[B[2]], B[6].at[pl.ds(0, sub)], B[9])
                pltpu.async_copy(xr0_hbm.at[B[3]], B[6].at[pl.ds(sub, sub)], B[9])

            @pl.when(cid == 1)
            def _():
                pltpu.async_copy(xl1_hbm.at[B[0]], B[5].at[pl.ds(0, sub)], B[9])
                pltpu.async_copy(xl1_hbm.at[B[1]], B[5].at[pl.ds(sub, sub)], B[9])
                pltpu.async_copy(xr1_hbm.at[B[2]], B[6].at[pl.ds(0, sub)], B[9])
                pltpu.async_copy(xr1_hbm.at[B[3]], B[6].at[pl.ds(sub, sub)], B[9])

        def wait_gather(B):
            pltpu.make_async_copy(
                xl0_hbm.at[B[0]], B[5].at[pl.ds(0, sub)], B[9]).wait()
            pltpu.make_async_copy(
                xl0_hbm.at[B[1]], B[5].at[pl.ds(sub, sub)], B[9]).wait()
            pltpu.make_async_copy(
                xr0_hbm.at[B[2]], B[6].at[pl.ds(0, sub)], B[9]).wait()
            pltpu.make_async_copy(
                xr0_hbm.at[B[3]], B[6].at[pl.ds(sub, sub)], B[9]).wait()

        def compute_scatter(B):
            ewv, xlr, xrr, valb = B[4], B[5], B[6], B[7]
            for k in range(nsub):
                for gg in range(sub // 16):
                    r0 = k * sub + gg * 16
                    rows = lax.iota(jnp.int32, 16) + r0
                    ewg = ewv[pl.ds(r0, 16)]
                    a0 = jnp.zeros((16,), jnp.float32)
                    xls = []
                    for comp in range(hid):
                        cix = jnp.full((16,), comp, jnp.int32)
                        xlc = plsc.load_gather(xlr, [rows, cix])
                        xrc = plsc.load_gather(xrr, [rows, cix])
                        xls.append(xlc)
                        t = xlc + xrc + ewg * wes_v[comp]
                        t = jnp.maximum(t, 0.2 * t)
                        a0 = a0 + t * atts_v[comp]
                    ex = jnp.exp(jnp.clip(a0, -60.0, 60.0))
                    for comp in range(hid):
                        cix = jnp.full((16,), comp, jnp.int32)
                        plsc.store_scatter(valb, [rows, cix], xls[comp] * ex)
                    plsc.store_scatter(
                        valb, [rows, jnp.full((16,), hid, jnp.int32)], ex)
            cp1 = pltpu.async_copy(
                valb.at[pl.ds(0, sub)], acc_sh.at[B[2]], B[8], add=True)
            cp2 = pltpu.async_copy(
                valb.at[pl.ds(sub, sub)], acc_sh.at[B[3]], B[9], add=True)
            cp1.wait()
            cp2.wait()

        issue_edge(0, bufs[0])
        wait_edge(bufs[0])
        issue_gather(bufs[0])
        issue_edge(1, bufs[1])

        def step(ci, B, Bo):
            @pl.when(ci + 1 < nch)
            def _():
                wait_edge(Bo)
                issue_gather(Bo)

            wait_gather(B)
            compute_scatter(B)

            @pl.when(ci + 2 < nch)
            def _():
                issue_edge(ci + 2, B)

        def chunk_body(ci, _):
            @pl.when(ci % 2 == 0)
            def _():
                step(ci, bufs[0], bufs[1])

            @pl.when(ci % 2 == 1)
            def _():
                step(ci, bufs[1], bufs[0])

            return 0

        lax.fori_loop(0, nch, chunk_body, 0)

        plsc.subcore_barrier()
        ostart = cid * n + start

        @pl.when(sid < ns - 1)
        def _():
            pltpu.sync_copy(acc_sh.at[pl.ds(start, stripe)],
                            out_hbm.at[pl.ds(ostart, stripe)])

        @pl.when(sid == ns - 1)
        def _():
            pltpu.sync_copy(acc_sh.at[pl.ds(start, last)],
                            out_hbm.at[pl.ds(ostart, last)])

    def bufset():
        return [
            pltpu.VMEM((sub,), jnp.int32),
            pltpu.VMEM((sub,), jnp.int32),
            pltpu.VMEM((sub,), jnp.int32),
            pltpu.VMEM((sub,), jnp.int32),
            pltpu.VMEM((cb,), jnp.float32),
            pltpu.VMEM((cb, _HW), jnp.float32),
            pltpu.VMEM((cb, _HW), jnp.float32),
            pltpu.VMEM((cb, _HW), jnp.float32),
        ]

    return pl.kernel(
        body,
        out_type=jax.ShapeDtypeStruct((2 * n, _HW), jnp.float32),
        mesh=mesh,
        compiler_params=pltpu.CompilerParams(
            needs_layout_passes=False, use_tc_tiling_on_sc=False),
        scratch_types=[
            pltpu.VMEM_SHARED((n, _HW), jnp.float32),
            pltpu.VMEM((_HW, 16), jnp.float32),
            pltpu.VMEM((_HW, 16), jnp.float32),
            *bufset(),
            *bufset(),
            pltpu.VMEM((zr, _HW), jnp.float32),
            pltpu.SemaphoreType.DMA,
            pltpu.SemaphoreType.DMA,
            pltpu.SemaphoreType.DMA,
            pltpu.SemaphoreType.DMA,
        ],
    )


def _post_body(acc_ref, gb0_ref, gb1_ref, W3a_ref, W3b_ref, b3_ref, W4_ref,
               b4_ref, out_ref):
    o0 = acc_ref[0]
    o1 = acc_ref[1]
    tn = o0.shape[0]
    d0 = jnp.broadcast_to(o0[:, 10:11], (tn, _HW)) + 1e-16
    d1 = jnp.broadcast_to(o1[:, 10:11], (tn, _HW)) + 1e-16
    g0 = o0 / d0 + gb0_ref[...]
    g1 = o1 / d1 + gb1_ref[...]
    h = (jnp.dot(g0, W3a_ref[...], preferred_element_type=jnp.float32)
         + jnp.dot(g1, W3b_ref[...], preferred_element_type=jnp.float32)
         + b3_ref[...])
    out_ref[...] = jnp.dot(h, W4_ref[...], preferred_element_type=jnp.float32) + b4_ref[...]


def _run_post(acc, gb0, gb1, W3a, W3b, b3, W4, b4, tn):
    n = acc.shape[1]
    ch = W4.shape[1]
    grid = n // tn

    def full(a):
        shp = a.shape
        return pl.BlockSpec(shp, lambda i, _s=len(shp): (0,) * _s)

    return pl.pallas_call(
        _post_body,
        grid=(grid,),
        in_specs=[pl.BlockSpec((2, tn, _HW), lambda i: (0, i, 0)), full(gb0),
                  full(gb1), full(W3a), full(W3b), full(b3), full(W4),
                  full(b4)],
        out_specs=pl.BlockSpec((tn, ch), lambda i: (i, 0)),
        out_shape=jax.ShapeDtypeStruct((n, ch), jnp.float32),
    )(acc, gb0, gb1, W3a, W3b, b3, W4, b4)


def kernel(signal, edge_index, edge_weight, mask, W1, b1, gamma, beta,
           run_mean, run_var, W2, b2, Wl, bl, Wr, br, We, att, gat_bias,
           W3, b3, W4, b4):
    f32 = jnp.float32
    n, ch = signal.shape
    e = edge_index.shape[1]
    hid = W2.shape[1]
    hc = Wl.shape[1]
    nheads = att.shape[0]
    hd = hc // nheads

    nm = jnp.logical_not(mask).astype(f32)[:, None]
    src = edge_index[0]
    dst = edge_index[1]
    ew = edge_weight[:, 0]

    def headw(W, h):
        return jnp.zeros((hid, _HW), f32).at[:, :hd].set(W[:, h * hd:(h + 1) * hd])

    def headb(v, h):
        return jnp.zeros((1, _HW), f32).at[0, :hd].set(v[h * hd:(h + 1) * hd])

    wl0, wl1 = headw(Wl, 0), headw(Wl, 1)
    wr0, wr1 = headw(Wr, 0), headw(Wr, 1)
    bl0, bl1 = headb(bl, 0), headb(bl, 1)
    br0, br1 = headb(br, 0), headb(br, 1)

    wesh = jnp.zeros((nheads, _HW, 16), f32)
    attsh = jnp.zeros((nheads, _HW, 16), f32)
    for h in range(nheads):
        wesh = wesh.at[h, :hd, :].set(
            jnp.broadcast_to(We[0, h * hd:(h + 1) * hd][:, None], (hd, 16)))
        attsh = attsh.at[h, :hd, :].set(
            jnp.broadcast_to(att[h][:, None], (hd, 16)))

    gb0, gb1 = headb(gat_bias, 0), headb(gat_bias, 1)
    W3a = jnp.zeros((_HW, W3.shape[1]), f32).at[:hd, :].set(W3[:hd, :])
    W3b = jnp.zeros((_HW, W3.shape[1]), f32).at[:hd, :].set(W3[hd:, :])

    tn = 2000
    xl0, xl1, xr0, xr1 = _run_pre(
        signal, nm, W1, b1.reshape(1, -1), gamma.reshape(1, -1),
        beta.reshape(1, -1), run_mean.reshape(1, -1), run_var.reshape(1, -1),
        W2, b2.reshape(1, -1), wl0, bl0, wl1, bl1, wr0, br0, wr1, br1, tn)

    acc = _make_sc_edge(n, e, hd)(
        xl0, xl1, xr0, xr1, src, dst, ew,
        wesh[0], wesh[1], attsh[0], attsh[1])

    return _run_post(acc.reshape(2, n, _HW), gb0, gb1, W3a, W3b,
                     b3.reshape(1, -1), W4, b4.reshape(1, -1), tn)
